# Initial kernel scaffold; baseline (speedup 1.0000x reference)
import jax
import jax.numpy as jnp
from jax.experimental import pallas as pl


def kernel(sparse_fea, W, b):
    raise NotImplementedError("write your pallas kernel here")


# speedup vs baseline: 32.2830x; 32.2830x over previous
import functools
import jax
import jax.numpy as jnp
from jax import lax
from jax.experimental import pallas as pl
from jax.experimental.pallas import tpu as pltpu
from jax.experimental.pallas import tpu_sc as plsc

BS, C, N = 8, 128, 2048
BLK = 256
NB = N // BLK


def _prep_body(xt_full_ref, xt_blk_ref, w_ref, b2_ref,
               at_ref, br_ref, i1_ref, i2_ref):
    bi = pl.program_id(0)
    xt = xt_full_ref[0]
    xb = xt_blk_ref[0]
    w1 = w_ref[:C, :]
    w2 = w_ref[C:, :]

    inner = lax.dot_general(xb, xt, (((0,), (0,)), ((), ())),
                            preferred_element_type=jnp.float32)
    sqj = jnp.sum(xt * xt, axis=0, keepdims=True)
    score = sqj - 2.0 * inner

    iota = lax.broadcasted_iota(jnp.int32, (BLK, N), 1)
    m1 = jnp.min(score, axis=1, keepdims=True)
    a1 = jnp.min(jnp.where(score == m1, iota, N), axis=1, keepdims=True)
    score2 = jnp.where(iota == a1, 1e30, score)
    m2 = jnp.min(score2, axis=1, keepdims=True)
    a2 = jnp.min(jnp.where(score2 == m2, iota, N), axis=1, keepdims=True)

    gbase = bi * N
    i1_ref[0] = a1 + gbase
    i2_ref[0] = a2 + gbase

    wd = w1 - w2
    at = lax.dot_general(wd, xb, (((0,), (0,)), ((), ())),
                         preferred_element_type=jnp.float32)
    at_ref[0] = at + b2_ref[...]
    br_ref[...] = lax.dot_general(xb, w2, (((0,), (0,)), ((), ())),
                                  preferred_element_type=jnp.float32)


_prep_call = pl.pallas_call(
    _prep_body,
    grid=(BS, NB),
    in_specs=[
        pl.BlockSpec((1, C, N), lambda bi, ii: (bi, 0, 0)),
        pl.BlockSpec((1, C, BLK), lambda bi, ii: (bi, 0, ii)),
        pl.BlockSpec((2 * C, C), lambda bi, ii: (0, 0)),
        pl.BlockSpec((C, 1), lambda bi, ii: (0, 0)),
    ],
    out_specs=[
        pl.BlockSpec((1, C, BLK), lambda bi, ii: (bi, 0, ii)),
        pl.BlockSpec((BLK, C), lambda bi, ii: (bi * NB + ii, 0)),
        pl.BlockSpec((1, BLK, 1), lambda bi, ii: (bi * NB + ii, 0, 0)),
        pl.BlockSpec((1, BLK, 1), lambda bi, ii: (bi * NB + ii, 0, 0)),
    ],
    out_shape=[
        jax.ShapeDtypeStruct((BS, C, N), jnp.float32),
        jax.ShapeDtypeStruct((BS * N, C), jnp.float32),
        jax.ShapeDtypeStruct((BS * NB, BLK, 1), jnp.int32),
        jax.ShapeDtypeStruct((BS * NB, BLK, 1), jnp.int32),
    ],
)


def _combine_body(at_ref, m_ref, o_ref):
    mt = m_ref[...].T
    h = at_ref[0] + mt
    o_ref[0] = jnp.where(h > 0, h, 0.2 * h)


_combine_call = pl.pallas_call(
    _combine_body,
    grid=(BS, NB),
    in_specs=[
        pl.BlockSpec((1, C, BLK), lambda bi, ii: (bi, 0, ii)),
        pl.BlockSpec((BLK, C), lambda bi, ii: (bi * NB + ii, 0)),
    ],
    out_specs=pl.BlockSpec((1, C, BLK), lambda bi, ii: (bi, 0, ii)),
    out_shape=jax.ShapeDtypeStruct((BS, C, N), jnp.float32),
)


_NC, _NS, _L = 2, 16, 16
NW = _NC * _NS
ROWS_PER_W = (BS * N) // NW
CB = 128
NCHUNK = ROWS_PER_W // CB

@functools.cache
def _make_sc_gather_max():
    mesh = plsc.VectorSubcoreMesh(core_axis_name="c", subcore_axis_name="s")

    @functools.partial(
        pl.kernel,
        mesh=mesh,
        out_type=jax.ShapeDtypeStruct((BS * N, C), jnp.float32),
        scratch_types=[
            pltpu.VMEM((CB,), jnp.int32),
            pltpu.VMEM((CB,), jnp.int32),
            pltpu.VMEM((CB, C), jnp.float32),
            pltpu.VMEM((CB, C), jnp.float32),
            pltpu.SemaphoreType.DMA,
            pltpu.SemaphoreType.DMA,
        ],
    )
    def sc_gather_max(br_hbm, i1_hbm, i2_hbm, out_hbm,
                      i1_v, i2_v, g1, g2, s1, s2):
        wid = lax.axis_index("s") * _NC + lax.axis_index("c")
        base = wid * ROWS_PER_W
        for ci in range(NCHUNK):
            off = base + ci * CB
            pltpu.sync_copy(i1_hbm.at[pl.ds(off, CB)], i1_v)
            pltpu.sync_copy(i2_hbm.at[pl.ds(off, CB)], i2_v)
            c1 = pltpu.async_copy(br_hbm.at[i1_v], g1, s1)
            c2 = pltpu.async_copy(br_hbm.at[i2_v], g2, s2)
            c1.wait()
            c2.wait()

            def row_body(r, carry):
                for k in range(C // _L):
                    sl = pl.ds(k * _L, _L)
                    g1[r, sl] = jnp.maximum(g1[r, sl], g2[r, sl])
                return carry

            lax.fori_loop(0, CB, row_body, 0)
            pltpu.sync_copy(g1, out_hbm.at[pl.ds(off, CB)])

    return sc_gather_max


def kernel(sparse_fea, W, b):
    b2 = b.reshape(C, 1)
    at, brows, i1, i2 = _prep_call(sparse_fea, sparse_fea, W, b2)
    m = _make_sc_gather_max()(brows, i1.reshape(BS * N), i2.reshape(BS * N))
    return _combine_call(at, m)


# speedup vs baseline: 35.1629x; 1.0892x over previous
import functools
import jax
import jax.numpy as jnp
from jax import lax
from jax.experimental import pallas as pl
from jax.experimental.pallas import tpu as pltpu
from jax.experimental.pallas import tpu_sc as plsc

BS, C, N = 8, 128, 2048
BLK = 512
NB = N // BLK


def _prep_body(xt_full_ref, xt_blk_ref, w_ref, b2_ref,
               at_ref, br_ref, i1_ref, i2_ref):
    bi = pl.program_id(0)
    xt = xt_full_ref[0]
    xb = xt_blk_ref[0]
    w1 = w_ref[:C, :]
    w2 = w_ref[C:, :]

    innerT = lax.dot_general(xt, xb, (((0,), (0,)), ((), ())),
                             preferred_element_type=jnp.float32)
    ones = jnp.ones((C, 1), dtype=jnp.float32)
    sqc = lax.dot_general(xt * xt, ones, (((0,), (0,)), ((), ())),
                          precision=lax.Precision.HIGHEST,
                          preferred_element_type=jnp.float32)
    score = sqc - 2.0 * innerT

    iota = lax.broadcasted_iota(jnp.int32, (N, BLK), 0)
    m1 = jnp.min(score, axis=0, keepdims=True)
    a1 = jnp.min(jnp.where(score == m1, iota, N), axis=0, keepdims=True)
    score2 = jnp.where(iota == a1, 1e30, score)
    m2 = jnp.min(score2, axis=0, keepdims=True)
    a2 = jnp.min(jnp.where(score2 == m2, iota, N), axis=0, keepdims=True)

    gbase = bi * N
    i1_ref[0] = a1 + gbase
    i2_ref[0] = a2 + gbase

    wd = w1 - w2
    at = lax.dot_general(wd, xb, (((0,), (0,)), ((), ())),
                         preferred_element_type=jnp.float32)
    at_ref[0] = at + b2_ref[...]
    br_ref[...] = lax.dot_general(xb, w2, (((0,), (0,)), ((), ())),
                                  preferred_element_type=jnp.float32)


_prep_call = pl.pallas_call(
    _prep_body,
    grid=(BS, NB),
    in_specs=[
        pl.BlockSpec((1, C, N), lambda bi, ii: (bi, 0, 0)),
        pl.BlockSpec((1, C, BLK), lambda bi, ii: (bi, 0, ii)),
        pl.BlockSpec((2 * C, C), lambda bi, ii: (0, 0)),
        pl.BlockSpec((C, 1), lambda bi, ii: (0, 0)),
    ],
    out_specs=[
        pl.BlockSpec((1, C, BLK), lambda bi, ii: (bi, 0, ii)),
        pl.BlockSpec((BLK, C), lambda bi, ii: (bi * NB + ii, 0)),
        pl.BlockSpec((1, 1, BLK), lambda bi, ii: (bi * NB + ii, 0, 0)),
        pl.BlockSpec((1, 1, BLK), lambda bi, ii: (bi * NB + ii, 0, 0)),
    ],
    out_shape=[
        jax.ShapeDtypeStruct((BS, C, N), jnp.float32),
        jax.ShapeDtypeStruct((BS * N, C), jnp.float32),
        jax.ShapeDtypeStruct((BS * NB, 1, BLK), jnp.int32),
        jax.ShapeDtypeStruct((BS * NB, 1, BLK), jnp.int32),
    ],
)


def _combine_body(at_ref, m_ref, o_ref):
    mt = m_ref[0].T
    h = at_ref[0] + mt
    o_ref[0] = jnp.where(h > 0, h, 0.2 * h)


_combine_call = pl.pallas_call(
    _combine_body,
    grid=(BS,),
    in_specs=[
        pl.BlockSpec((1, C, N), lambda bi: (bi, 0, 0)),
        pl.BlockSpec((1, N, C), lambda bi: (bi, 0, 0)),
    ],
    out_specs=pl.BlockSpec((1, C, N), lambda bi: (bi, 0, 0)),
    out_shape=jax.ShapeDtypeStruct((BS, C, N), jnp.float32),
)


_NC, _NS, _L = 2, 16, 16
NW = _NC * _NS
ROWS_PER_W = (BS * N) // NW
CB = 128
NCHUNK = ROWS_PER_W // CB


@functools.cache
def _make_sc_gather_max():
    mesh = plsc.VectorSubcoreMesh(core_axis_name="c", subcore_axis_name="s")

    @functools.partial(
        pl.kernel,
        mesh=mesh,
        out_type=jax.ShapeDtypeStruct((BS * N, C), jnp.float32),
        scratch_types=[
            pltpu.VMEM((CB,), jnp.int32),
            pltpu.VMEM((CB,), jnp.int32),
            pltpu.VMEM((CB, C), jnp.float32),
            pltpu.VMEM((CB, C), jnp.float32),
            pltpu.SemaphoreType.DMA,
            pltpu.SemaphoreType.DMA,
        ],
    )
    def sc_gather_max(br_hbm, i1_hbm, i2_hbm, out_hbm,
                      i1_v, i2_v, g1, g2, s1, s2):
        wid = lax.axis_index("s") * _NC + lax.axis_index("c")
        base = wid * ROWS_PER_W
        for ci in range(NCHUNK):
            off = base + ci * CB
            pltpu.sync_copy(i1_hbm.at[pl.ds(off, CB)], i1_v)
            pltpu.sync_copy(i2_hbm.at[pl.ds(off, CB)], i2_v)
            c1 = pltpu.async_copy(br_hbm.at[i1_v], g1, s1)
            c2 = pltpu.async_copy(br_hbm.at[i2_v], g2, s2)
            c1.wait()
            c2.wait()

            def row_body(r, carry):
                for k in range(C // _L):
                    sl = pl.ds(k * _L, _L)
                    g1[r, sl] = jnp.maximum(g1[r, sl], g2[r, sl])
                return carry

            lax.fori_loop(0, CB, row_body, 0)
            pltpu.sync_copy(g1, out_hbm.at[pl.ds(off, CB)])

    return sc_gather_max


def kernel(sparse_fea, W, b):
    b2 = b.reshape(C, 1)
    at, brows, i1, i2 = _prep_call(sparse_fea, sparse_fea, W, b2)
    m = _make_sc_gather_max()(brows, i1.reshape(BS * N), i2.reshape(BS * N))
    return _combine_call(at, m.reshape(BS, N, C))


# speedup vs baseline: 51.7039x; 1.4704x over previous
import functools
import jax
import jax.numpy as jnp
from jax import lax
from jax.experimental import pallas as pl
from jax.experimental.pallas import tpu as pltpu
from jax.experimental.pallas import tpu_sc as plsc

BS, C, N = 8, 128, 2048
BLK = 512
NB = N // BLK


def _prep_body(xt_full_ref, xt_blk_ref, w_ref, b2_ref,
               at_ref, br_ref, i2_ref):
    bi = pl.program_id(0)
    xt = xt_full_ref[0]
    xb = xt_blk_ref[0]
    w1 = w_ref[:C, :]
    w2 = w_ref[C:, :]

    innerT = lax.dot_general(xt, xb * -2.0, (((0,), (0,)), ((), ())),
                             preferred_element_type=jnp.float32)
    sq_row = jnp.sum(xt * xt, axis=0, keepdims=True)
    sqc = sq_row.T
    score = sqc + innerT

    fiota = lax.broadcasted_iota(jnp.int32, (N, BLK), 0).astype(jnp.float32)
    m1 = jnp.min(score, axis=0, keepdims=True)
    score2 = jnp.where(score == m1, 1e30, score)
    m2 = jnp.min(score2, axis=0, keepdims=True)
    a2f = jnp.min(jnp.where(score2 == m2, fiota, float(N)),
                  axis=0, keepdims=True)

    i2_ref[0] = a2f.astype(jnp.int32) + bi * N

    wd = w1 - w2
    at = lax.dot_general(wd, xb, (((0,), (0,)), ((), ())),
                         preferred_element_type=jnp.float32)
    at_ref[0] = at + b2_ref[...]
    br_ref[...] = lax.dot_general(xb, w2, (((0,), (0,)), ((), ())),
                                  preferred_element_type=jnp.float32)


_prep_call = pl.pallas_call(
    _prep_body,
    grid=(BS, NB),
    in_specs=[
        pl.BlockSpec((1, C, N), lambda bi, ii: (bi, 0, 0)),
        pl.BlockSpec((1, C, BLK), lambda bi, ii: (bi, 0, ii)),
        pl.BlockSpec((2 * C, C), lambda bi, ii: (0, 0)),
        pl.BlockSpec((C, 1), lambda bi, ii: (0, 0)),
    ],
    out_specs=[
        pl.BlockSpec((1, C, BLK), lambda bi, ii: (bi, 0, ii)),
        pl.BlockSpec((BLK, C), lambda bi, ii: (bi * NB + ii, 0)),
        pl.BlockSpec((1, 1, BLK), lambda bi, ii: (bi * NB + ii, 0, 0)),
    ],
    out_shape=[
        jax.ShapeDtypeStruct((BS, C, N), jnp.float32),
        jax.ShapeDtypeStruct((BS * N, C), jnp.float32),
        jax.ShapeDtypeStruct((BS * NB, 1, BLK), jnp.int32),
    ],
)


def _combine_body(at_ref, m_ref, o_ref):
    mt = m_ref[0].T
    h = at_ref[0] + mt
    o_ref[0] = jnp.where(h > 0, h, 0.2 * h)


_combine_call = pl.pallas_call(
    _combine_body,
    grid=(BS,),
    in_specs=[
        pl.BlockSpec((1, C, N), lambda bi: (bi, 0, 0)),
        pl.BlockSpec((1, N, C), lambda bi: (bi, 0, 0)),
    ],
    out_specs=pl.BlockSpec((1, C, N), lambda bi: (bi, 0, 0)),
    out_shape=jax.ShapeDtypeStruct((BS, C, N), jnp.float32),
)


_NC, _NS, _L = 2, 16, 16
NW = _NC * _NS
ROWS_PER_W = (BS * N) // NW
CB = 128
NCHUNK = ROWS_PER_W // CB


@functools.cache
def _make_sc_gather_max():
    mesh = plsc.VectorSubcoreMesh(core_axis_name="c", subcore_axis_name="s")

    @functools.partial(
        pl.kernel,
        mesh=mesh,
        out_type=jax.ShapeDtypeStruct((BS * N, C), jnp.float32),
        scratch_types=[
            pltpu.VMEM((CB,), jnp.int32),
            pltpu.VMEM((CB, C), jnp.float32),
            pltpu.VMEM((CB, C), jnp.float32),
            pltpu.SemaphoreType.DMA,
        ],
    )
    def sc_gather_max(br_hbm, i2_hbm, out_hbm, i2_v, g1, g2, s2):
        wid = lax.axis_index("s") * _NC + lax.axis_index("c")
        base = wid * ROWS_PER_W
        for ci in range(NCHUNK):
            off = base + ci * CB
            pltpu.sync_copy(i2_hbm.at[pl.ds(off, CB)], i2_v)
            c2 = pltpu.async_copy(br_hbm.at[i2_v], g2, s2)
            pltpu.sync_copy(br_hbm.at[pl.ds(off, CB)], g1)
            c2.wait()

            def row_body(r, carry):
                for k in range(C // _L):
                    sl = pl.ds(k * _L, _L)
                    g1[r, sl] = jnp.maximum(g1[r, sl], g2[r, sl])
                return carry

            lax.fori_loop(0, CB, row_body, 0)
            pltpu.sync_copy(g1, out_hbm.at[pl.ds(off, CB)])

    return sc_gather_max


def kernel(sparse_fea, W, b):
    b2 = b.reshape(C, 1)
    at, brows, i2 = _prep_call(sparse_fea, sparse_fea, W, b2)
    m = _make_sc_gather_max()(brows, i2.reshape(BS * N))
    return _combine_call(at, m.reshape(BS, N, C))


# speedup vs baseline: 54.1825x; 1.0479x over previous
import functools
import jax
import jax.numpy as jnp
from jax import lax
from jax.experimental import pallas as pl
from jax.experimental.pallas import tpu as pltpu
from jax.experimental.pallas import tpu_sc as plsc

BS, C, N = 8, 128, 2048
BLK = 512
NB = N // BLK


def _prep_body(xt_full_ref, xt_blk_ref, w_ref, b2_ref, iota_ref,
               at_ref, bt_ref, br_ref, i2_ref):
    bi = pl.program_id(0)
    xt = xt_full_ref[0]
    xb = xt_blk_ref[0]
    w1 = w_ref[:C, :]
    w2 = w_ref[C:, :]

    innerT = lax.dot_general(xt, xb * -2.0, (((0,), (0,)), ((), ())),
                             preferred_element_type=jnp.float32)
    sq_row = jnp.sum(xt * xt, axis=0, keepdims=True)
    sqc = sq_row.T
    score = sqc + innerT

    fiota = iota_ref[...]
    m1 = jnp.min(score, axis=0, keepdims=True)
    score2 = jnp.where(score == m1, 1e30, score)
    m2 = jnp.min(score2, axis=0, keepdims=True)
    a2f = jnp.min(jnp.where(score2 == m2, fiota, float(N)),
                  axis=0, keepdims=True)

    i2_ref[0] = a2f.astype(jnp.int32) + bi * N

    wd = w1 - w2
    at = lax.dot_general(wd, xb, (((0,), (0,)), ((), ())),
                         preferred_element_type=jnp.float32)
    at_ref[0] = at + b2_ref[...]
    bt_ref[0] = lax.dot_general(w2, xb, (((0,), (0,)), ((), ())),
                                preferred_element_type=jnp.float32)
    br_ref[...] = lax.dot_general(xb, w2, (((0,), (0,)), ((), ())),
                                  preferred_element_type=jnp.float32)


_prep_call = pl.pallas_call(
    _prep_body,
    grid=(BS, NB),
    in_specs=[
        pl.BlockSpec((1, C, N), lambda bi, ii: (bi, 0, 0)),
        pl.BlockSpec((1, C, BLK), lambda bi, ii: (bi, 0, ii)),
        pl.BlockSpec((2 * C, C), lambda bi, ii: (0, 0)),
        pl.BlockSpec((C, 1), lambda bi, ii: (0, 0)),
        pl.BlockSpec((N, 1), lambda bi, ii: (0, 0)),
    ],
    out_specs=[
        pl.BlockSpec((1, C, BLK), lambda bi, ii: (bi, 0, ii)),
        pl.BlockSpec((1, C, BLK), lambda bi, ii: (bi, 0, ii)),
        pl.BlockSpec((BLK, C), lambda bi, ii: (bi * NB + ii, 0)),
        pl.BlockSpec((1, 1, BLK), lambda bi, ii: (bi * NB + ii, 0, 0)),
    ],
    out_shape=[
        jax.ShapeDtypeStruct((BS, C, N), jnp.float32),
        jax.ShapeDtypeStruct((BS, C, N), jnp.float32),
        jax.ShapeDtypeStruct((BS * N, C), jnp.float32),
        jax.ShapeDtypeStruct((BS * NB, 1, BLK), jnp.int32),
    ],
)


def _combine_body(at_ref, bt_ref, m_ref, o_ref):
    mt = m_ref[0].T
    h = at_ref[0] + jnp.maximum(bt_ref[0], mt)
    o_ref[0] = jnp.where(h > 0, h, 0.2 * h)


_combine_call = pl.pallas_call(
    _combine_body,
    grid=(BS,),
    in_specs=[
        pl.BlockSpec((1, C, N), lambda bi: (bi, 0, 0)),
        pl.BlockSpec((1, C, N), lambda bi: (bi, 0, 0)),
        pl.BlockSpec((1, N, C), lambda bi: (bi, 0, 0)),
    ],
    out_specs=pl.BlockSpec((1, C, N), lambda bi: (bi, 0, 0)),
    out_shape=jax.ShapeDtypeStruct((BS, C, N), jnp.float32),
)


_NC, _NS, _L = 2, 16, 16
NW = _NC * _NS
ROWS_PER_W = (BS * N) // NW
CB = 128
NCHUNK = ROWS_PER_W // CB


@functools.cache
def _make_sc_gather_max():
    mesh = plsc.VectorSubcoreMesh(core_axis_name="c", subcore_axis_name="s")

    @functools.partial(
        pl.kernel,
        mesh=mesh,
        out_type=jax.ShapeDtypeStruct((BS * N, C), jnp.float32),
        scratch_types=[
            pltpu.VMEM((CB,), jnp.int32),
            pltpu.VMEM((CB,), jnp.int32),
            pltpu.VMEM((CB, C), jnp.float32),
            pltpu.VMEM((CB, C), jnp.float32),
            pltpu.SemaphoreType.DMA,
            pltpu.SemaphoreType.DMA,
        ],
    )
    def sc_gather(br_hbm, i2_hbm, out_hbm, iva, ivb, ga, gb, sa, sb):
        wid = lax.axis_index("s") * _NC + lax.axis_index("c")
        base = wid * ROWS_PER_W
        idx_v = (iva, ivb)
        g_v = (ga, gb)
        sem = (sa, sb)
        copies = [None, None]
        pltpu.sync_copy(i2_hbm.at[pl.ds(base, CB)], iva)
        copies[0] = pltpu.async_copy(br_hbm.at[iva], ga, sa)
        for ci in range(NCHUNK):
            cur = ci % 2
            nxt = (ci + 1) % 2
            if ci + 1 < NCHUNK:
                off_n = base + (ci + 1) * CB
                pltpu.sync_copy(i2_hbm.at[pl.ds(off_n, CB)], idx_v[nxt])
                copies[nxt] = pltpu.async_copy(
                    br_hbm.at[idx_v[nxt]], g_v[nxt], sem[nxt])
            copies[cur].wait()
            pltpu.sync_copy(g_v[cur], out_hbm.at[pl.ds(base + ci * CB, CB)])

    return sc_gather


def kernel(sparse_fea, W, b):
    iota_col = jnp.arange(N, dtype=jnp.float32).reshape(N, 1)
    b2 = b.reshape(C, 1)
    at, bt, brows, i2 = _prep_call(sparse_fea, sparse_fea, W, b2, iota_col)
    m = _make_sc_gather_max()(brows, i2.reshape(BS * N))
    return _combine_call(at, bt, m.reshape(BS, N, C))


# speedup vs baseline: 56.5930x; 1.0445x over previous
import functools
import jax
import jax.numpy as jnp
from jax import lax
from jax.experimental import pallas as pl
from jax.experimental.pallas import tpu as pltpu
from jax.experimental.pallas import tpu_sc as plsc

BS, C, N = 8, 128, 2048
BLK = 512
NB = N // BLK


def _prep_body(xt_full_ref, xt_blk_ref, w_ref, iota_ref, br_ref, i2_ref):
    bi = pl.program_id(0)
    xt = xt_full_ref[0]
    xb = xt_blk_ref[0]
    w2 = w_ref[C:, :]

    innerT = lax.dot_general(xt, xb * -2.0, (((0,), (0,)), ((), ())),
                             preferred_element_type=jnp.float32)
    sq_row = jnp.sum(xt * xt, axis=0, keepdims=True)
    sqc = sq_row.T
    score = sqc + innerT

    fiota = iota_ref[...]
    m1 = jnp.min(score, axis=0, keepdims=True)
    score2 = jnp.where(score == m1, 1e30, score)
    m2 = jnp.min(score2, axis=0, keepdims=True)
    a2f = jnp.min(jnp.where(score2 == m2, fiota, float(N)),
                  axis=0, keepdims=True)

    i2_ref[0] = a2f.astype(jnp.int32) + bi * N

    br_ref[...] = lax.dot_general(xb, w2, (((0,), (0,)), ((), ())),
                                  preferred_element_type=jnp.float32)


_prep_call = pl.pallas_call(
    _prep_body,
    grid=(BS, NB),
    in_specs=[
        pl.BlockSpec((1, C, N), lambda bi, ii: (bi, 0, 0)),
        pl.BlockSpec((1, C, BLK), lambda bi, ii: (bi, 0, ii)),
        pl.BlockSpec((2 * C, C), lambda bi, ii: (0, 0)),
        pl.BlockSpec((N, 1), lambda bi, ii: (0, 0)),
    ],
    out_specs=[
        pl.BlockSpec((BLK, C), lambda bi, ii: (bi * NB + ii, 0)),
        pl.BlockSpec((1, 1, BLK), lambda bi, ii: (bi * NB + ii, 0, 0)),
    ],
    out_shape=[
        jax.ShapeDtypeStruct((BS * N, C), jnp.float32),
        jax.ShapeDtypeStruct((BS * NB, 1, BLK), jnp.int32),
    ],
)


def _combine_body(xt_ref, w_ref, b2_ref, m_ref, o_ref):
    xt = xt_ref[0]
    w1 = w_ref[:C, :]
    w2 = w_ref[C:, :]
    at = lax.dot_general(w1 - w2, xt, (((0,), (0,)), ((), ())),
                         preferred_element_type=jnp.float32)
    bt = lax.dot_general(w2, xt, (((0,), (0,)), ((), ())),
                         preferred_element_type=jnp.float32)
    mt = m_ref[0].T
    h = (at + b2_ref[...]) + jnp.maximum(bt, mt)
    o_ref[0] = jnp.where(h > 0, h, 0.2 * h)


_combine_call = pl.pallas_call(
    _combine_body,
    grid=(BS,),
    in_specs=[
        pl.BlockSpec((1, C, N), lambda bi: (bi, 0, 0)),
        pl.BlockSpec((2 * C, C), lambda bi: (0, 0)),
        pl.BlockSpec((C, 1), lambda bi: (0, 0)),
        pl.BlockSpec((1, N, C), lambda bi: (bi, 0, 0)),
    ],
    out_specs=pl.BlockSpec((1, C, N), lambda bi: (bi, 0, 0)),
    out_shape=jax.ShapeDtypeStruct((BS, C, N), jnp.float32),
)


_NC, _NS, _L = 2, 16, 16
NW = _NC * _NS
ROWS_PER_W = (BS * N) // NW
CB = 256
NCHUNK = ROWS_PER_W // CB


@functools.cache
def _make_sc_gather():
    mesh = plsc.VectorSubcoreMesh(core_axis_name="c", subcore_axis_name="s")

    @functools.partial(
        pl.kernel,
        mesh=mesh,
        out_type=jax.ShapeDtypeStruct((BS * N, C), jnp.float32),
        scratch_types=[
            pltpu.VMEM((CB,), jnp.int32),
            pltpu.VMEM((CB,), jnp.int32),
            pltpu.VMEM((CB, C), jnp.float32),
            pltpu.VMEM((CB, C), jnp.float32),
            pltpu.SemaphoreType.DMA,
            pltpu.SemaphoreType.DMA,
        ],
    )
    def sc_gather(br_hbm, i2_hbm, out_hbm, iva, ivb, ga, gb, sa, sb):
        wid = lax.axis_index("s") * _NC + lax.axis_index("c")
        base = wid * ROWS_PER_W
        idx_v = (iva, ivb)
        g_v = (ga, gb)
        sem = (sa, sb)
        copies = [None, None]
        pltpu.sync_copy(i2_hbm.at[pl.ds(base, CB)], iva)
        copies[0] = pltpu.async_copy(br_hbm.at[iva], ga, sa)
        for ci in range(NCHUNK):
            cur = ci % 2
            nxt = (ci + 1) % 2
            if ci + 1 < NCHUNK:
                off_n = base + (ci + 1) * CB
                pltpu.sync_copy(i2_hbm.at[pl.ds(off_n, CB)], idx_v[nxt])
                copies[nxt] = pltpu.async_copy(
                    br_hbm.at[idx_v[nxt]], g_v[nxt], sem[nxt])
            copies[cur].wait()
            pltpu.sync_copy(g_v[cur], out_hbm.at[pl.ds(base + ci * CB, CB)])

    return sc_gather


def kernel(sparse_fea, W, b):
    iota_col = jnp.arange(N, dtype=jnp.float32).reshape(N, 1)
    b2 = b.reshape(C, 1)
    brows, i2 = _prep_call(sparse_fea, sparse_fea, W, iota_col)
    m = _make_sc_gather()(brows, i2.reshape(BS * N))
    return _combine_call(sparse_fea, W, b2, m.reshape(BS, N, C))
